# baseline (speedup 1.0000x reference)
import functools

import jax
import jax.numpy as jnp
from jax import lax
from jax.experimental import pallas as pl
from jax.experimental.pallas import tpu as pltpu
from jax.experimental.pallas import tpu_sc as plsc

B, T, DIM = 2, 2048, 768
E, K, HID = 8, 2, 2048
INNER = 2 * HID
N = B * T
P = N * K
TM = 256
PADTOT = P + E * TM
GT = PADTOT // TM
NSUB = 16
NW = 32
CP = P // NW
GCH = 64
NB = 16
BP = P // NB


def _router_body(x_ref, rw_ref, rb_ref, eid_ref, w_ref):
    logits = jax.lax.dot_general(
        x_ref[...], rw_ref[...], (((1,), (1,)), ((), ())),
        preferred_element_type=jnp.float32,
    ) + rb_ref[...]
    idx = jax.lax.broadcasted_iota(jnp.int32, (N, E), 1)
    m1 = jnp.max(logits, axis=1, keepdims=True)
    a1 = jnp.min(jnp.where(logits == m1, idx, E), axis=1, keepdims=True)
    l2 = jnp.where(idx == a1, -jnp.inf, logits)
    m2 = jnp.max(l2, axis=1, keepdims=True)
    a2 = jnp.min(jnp.where(l2 == m2, idx, E), axis=1, keepdims=True)
    w1 = jax.nn.sigmoid(m1 - m2)
    eid_ref[...] = jnp.concatenate([a1, a2], axis=1)
    w_ref[...] = jnp.concatenate([w1, 1.0 - w1], axis=1)


def _router(xf, router_W, router_b):
    return pl.pallas_call(
        _router_body,
        out_shape=(
            jax.ShapeDtypeStruct((N, K), jnp.int32),
            jax.ShapeDtypeStruct((N, K), jnp.float32),
        ),
        in_specs=[
            pl.BlockSpec((N, DIM), lambda: (0, 0)),
            pl.BlockSpec((E, DIM), lambda: (0, 0)),
            pl.BlockSpec((E,), lambda: (0,)),
        ],
        out_specs=(
            pl.BlockSpec((N, K), lambda: (0, 0)),
            pl.BlockSpec((N, K), lambda: (0, 0)),
        ),
    )(xf, router_W, router_b)


def _bin_body(e_ref, pos_ref, gids_ref, tot_ref, run_ref):
    ph = pl.program_id(0)
    blk = pl.program_id(1)

    eb = e_ref[0]
    lanes8 = jax.lax.broadcasted_iota(jnp.int32, (1, E), 1)
    oh = (eb == lanes8).astype(jnp.float32)

    @pl.when(jnp.logical_and(ph == 0, blk == 0))
    def _init0():
        tot_ref[...] = jnp.zeros((1, 128), jnp.float32)

    @pl.when(ph == 0)
    def _count():
        tot_ref[:, :E] = tot_ref[:, :E] + jnp.sum(oh, axis=0, keepdims=True)
        pos_ref[...] = jnp.zeros((1, BP, 1), jnp.int32)

    @pl.when(jnp.logical_and(ph == 1, blk == 0))
    def _init1():
        run_ref[...] = jnp.zeros((1, 128), jnp.float32)

    @pl.when(ph == 1)
    def _rank():
        tot = tot_ref[:, :E]
        toti = tot.astype(jnp.int32)
        padcnt = ((toti + (TM - 1)) >> 8) << 8
        le8 = jax.lax.broadcasted_iota(jnp.int32, (E, E), 0) <= \
            jax.lax.broadcasted_iota(jnp.int32, (E, E), 1)
        incl = jax.lax.dot_general(
            padcnt.astype(jnp.float32), le8.astype(jnp.float32),
            (((1,), (0,)), ((), ())), preferred_element_type=jnp.float32)
        padoff = incl - padcnt.astype(jnp.float32)

        ir = jax.lax.broadcasted_iota(jnp.int32, (BP, BP), 0)
        ic = jax.lax.broadcasted_iota(jnp.int32, (BP, BP), 1)
        tri = (ic < ir).astype(jnp.float32)
        excl = jax.lax.dot_general(
            tri, oh, (((1,), (0,)), ((), ())),
            preferred_element_type=jnp.float32)

        slot = jnp.sum(oh * (padoff + run_ref[:, :E] + excl),
                       axis=1, keepdims=True)
        pos_ref[...] = slot.astype(jnp.int32).reshape(1, BP, 1)
        run_ref[:, :E] = run_ref[:, :E] + jnp.sum(oh, axis=0, keepdims=True)

        @pl.when(blk == 0)
        def _gids():
            ts = jax.lax.broadcasted_iota(jnp.int32, (1, 64), 1).astype(
                jnp.float32) * TM
            g = jnp.zeros((1, 64), jnp.float32)
            for e in range(E):
                incl_e = jax.lax.dot_general(
                    incl, (lanes8 == e).astype(jnp.float32),
                    (((1,), (1,)), ((), ())),
                    preferred_element_type=jnp.float32)
                g = g + jnp.where(ts >= incl_e, 1.0, 0.0)
            gids_ref[...] = jnp.minimum(g, E - 1).astype(jnp.int32)


def _binning(eT3):
    return pl.pallas_call(
        _bin_body,
        grid=(2, NB),
        out_shape=(
            jax.ShapeDtypeStruct((NB + 1, BP, 1), jnp.int32),
            jax.ShapeDtypeStruct((1, 64), jnp.int32),
        ),
        in_specs=[pl.BlockSpec((1, BP, 1), lambda ph, blk: (blk, 0, 0))],
        out_specs=(
            pl.BlockSpec((1, BP, 1),
                         lambda ph, blk: (jnp.where(ph == 0, NB, blk), 0, 0)),
            pl.BlockSpec((1, 64), lambda ph, blk: (0, 0)),
        ),
        scratch_shapes=[
            pltpu.VMEM((1, 128), jnp.float32),
            pltpu.VMEM((1, 128), jnp.float32),
        ],
        compiler_params=pltpu.CompilerParams(
            dimension_semantics=("arbitrary", "arbitrary"),
        ),
    )(eT3)


@functools.cache
def _mesh():
    return plsc.VectorSubcoreMesh(core_axis_name="c", subcore_axis_name="s",
                                  num_cores=2, num_subcores=NSUB)


def _sc_dispatch_body(pos_hbm, x_hbm, xdisp_hbm, pos_v, rows0, rows1,
                      lsem0, lsem1, ssem0, ssem1):
    wid = lax.axis_index("s") * 2 + lax.axis_index("c")
    tokbase = (wid % NSUB) * CP
    pltpu.sync_copy(pos_hbm.at[pl.ds(wid * (CP // GCH), CP // GCH)], pos_v)
    rows, lsem, ssem = [rows0, rows1], [lsem0, lsem1], [ssem0, ssem1]
    nch = CP // GCH
    loads, scats = [None] * nch, [None] * nch
    loads[0] = pltpu.async_copy(x_hbm.at[pl.ds(tokbase, GCH)], rows0, lsem0)
    for g in range(nch):
        b = g % 2
        loads[g].wait()
        scats[g] = pltpu.async_copy(rows[b], xdisp_hbm.at[pos_v.at[g]], ssem[b])
        if g + 1 < nch:
            if g >= 1:
                scats[g - 1].wait()
            loads[g + 1] = pltpu.async_copy(
                x_hbm.at[pl.ds(tokbase + (g + 1) * GCH, GCH)],
                rows[1 - b], lsem[1 - b])
    scats[nch - 2].wait()
    scats[nch - 1].wait()


@functools.cache
def _sc_dispatch():
    return pl.kernel(
        _sc_dispatch_body,
        out_type=jax.ShapeDtypeStruct((PADTOT, DIM), jnp.float32),
        mesh=_mesh(),
        compiler_params=pltpu.CompilerParams(needs_layout_passes=False),
        scratch_types=[
            pltpu.VMEM((CP // GCH, GCH), jnp.int32),
            pltpu.VMEM((GCH, DIM), jnp.float32),
            pltpu.VMEM((GCH, DIM), jnp.float32),
            pltpu.SemaphoreType.DMA,
            pltpu.SemaphoreType.DMA,
            pltpu.SemaphoreType.DMA,
            pltpu.SemaphoreType.DMA,
        ],
    )


def _gmm_body(gid_ref, x_ref, w1_ref, b1_ref, w2_ref, b2_ref, y_ref):
    del gid_ref
    xt = x_ref[...]
    y = jnp.zeros((TM, DIM), jnp.float32)
    CHUNK = 2048
    for j0 in range(0, HID, CHUNK):
        a = jnp.dot(xt, w1_ref[0, :, j0:j0 + CHUNK],
                    preferred_element_type=jnp.float32) + b1_ref[0, 0, j0:j0 + CHUNK]
        g = jnp.dot(xt, w1_ref[0, :, HID + j0:HID + j0 + CHUNK],
                    preferred_element_type=jnp.float32) + b1_ref[0, 0, HID + j0:HID + j0 + CHUNK]
        act = (a * jax.nn.sigmoid(a)) * g
        y = y + jnp.dot(act, w2_ref[0, j0:j0 + CHUNK, :],
                        preferred_element_type=jnp.float32)
    y_ref[...] = y + b2_ref[0, 0, :]


def _gmm(gids, xdisp, w1b, b1r, w2b, b2r):
    grid_spec = pltpu.PrefetchScalarGridSpec(
        num_scalar_prefetch=1,
        grid=(GT,),
        in_specs=[
            pl.BlockSpec((TM, DIM), lambda i, g: (i, 0)),
            pl.BlockSpec((1, DIM, INNER), lambda i, g: (g[i], 0, 0)),
            pl.BlockSpec((1, 1, INNER), lambda i, g: (g[i], 0, 0)),
            pl.BlockSpec((1, HID, DIM), lambda i, g: (g[i], 0, 0)),
            pl.BlockSpec((1, 1, DIM), lambda i, g: (g[i], 0, 0)),
        ],
        out_specs=pl.BlockSpec((TM, DIM), lambda i, g: (i, 0)),
    )
    return pl.pallas_call(
        _gmm_body,
        grid_spec=grid_spec,
        out_shape=jax.ShapeDtypeStruct((PADTOT, DIM), jnp.float32),
        compiler_params=pltpu.CompilerParams(
            dimension_semantics=("arbitrary",),
        ),
    )(gids, xdisp, w1b, b1r, w2b, b2r)


def _sc_combine_body(ydisp_hbm, pos_hbm, ysort_hbm, pos_v, rows0, rows1,
                     gsem0, gsem1, psem0, psem1):
    wid = lax.axis_index("s") * 2 + lax.axis_index("c")
    r0 = wid * CP
    pltpu.sync_copy(pos_hbm.at[pl.ds(wid * (CP // GCH), CP // GCH)], pos_v)
    rows, gsem, psem = [rows0, rows1], [gsem0, gsem1], [psem0, psem1]
    nch = CP // GCH
    gets, puts = [None] * nch, [None] * nch
    gets[0] = pltpu.async_copy(ydisp_hbm.at[pos_v.at[0]], rows0, gsem0)
    for g in range(nch):
        b = g % 2
        gets[g].wait()
        puts[g] = pltpu.async_copy(
            rows[b], ysort_hbm.at[pl.ds(r0 + g * GCH, GCH)], psem[b])
        if g + 1 < nch:
            if g >= 1:
                puts[g - 1].wait()
            gets[g + 1] = pltpu.async_copy(
                ydisp_hbm.at[pos_v.at[g + 1]], rows[1 - b], gsem[1 - b])
    puts[nch - 2].wait()
    puts[nch - 1].wait()


@functools.cache
def _sc_combine():
    return pl.kernel(
        _sc_combine_body,
        out_type=jax.ShapeDtypeStruct((P, DIM), jnp.float32),
        mesh=_mesh(),
        compiler_params=pltpu.CompilerParams(needs_layout_passes=False),
        scratch_types=[
            pltpu.VMEM((CP // GCH, GCH), jnp.int32),
            pltpu.VMEM((GCH, DIM), jnp.float32),
            pltpu.VMEM((GCH, DIM), jnp.float32),
            pltpu.SemaphoreType.DMA,
            pltpu.SemaphoreType.DMA,
            pltpu.SemaphoreType.DMA,
            pltpu.SemaphoreType.DMA,
        ],
    )


def _add_body(w_ref, y0_ref, y1_ref, o_ref):
    w = w_ref[...]
    o_ref[...] = w[:, 0:1] * y0_ref[0] + w[:, 1:2] * y1_ref[0]


def _pair_add(wts, ys3):
    NTB = 8
    return pl.pallas_call(
        _add_body,
        grid=(NTB,),
        out_shape=jax.ShapeDtypeStruct((N, DIM), jnp.float32),
        in_specs=[
            pl.BlockSpec((N // NTB, K), lambda i: (i, 0)),
            pl.BlockSpec((1, N // NTB, DIM), lambda i: (0, i, 0)),
            pl.BlockSpec((1, N // NTB, DIM), lambda i: (1, i, 0)),
        ],
        out_specs=pl.BlockSpec((N // NTB, DIM), lambda i: (i, 0)),
    )(wts, ys3, ys3)


@jax.jit
def kernel(x, router_W, router_b, W1, b1, W2, b2):
    xf = x.reshape(N, DIM)
    eids, wts = _router(xf, router_W, router_b)
    eT3 = jnp.transpose(eids).reshape(NB, BP, 1)
    pos3, gids64 = _binning(eT3)
    posf = pos3[:NB].reshape(P)
    xdisp = _sc_dispatch()(posf.reshape(NW * (CP // GCH), GCH), xf)
    ydisp = _gmm(gids64.reshape(64)[:GT], xdisp,
                 W1, b1.reshape(E, 1, INNER),
                 W2, b2.reshape(E, 1, DIM))
    ysorted = _sc_combine()(ydisp, posf.reshape(NW * (CP // GCH), GCH))
    out = _pair_add(wts, ysorted.reshape(K, N, DIM))
    return out.reshape(B, T, DIM)


# speedup vs baseline: 1.0591x; 1.0591x over previous
import functools

import jax
import jax.numpy as jnp
from jax import lax
from jax.experimental import pallas as pl
from jax.experimental.pallas import tpu as pltpu
from jax.experimental.pallas import tpu_sc as plsc

B, T, DIM = 2, 2048, 768
E, K, HID = 8, 2, 2048
INNER = 2 * HID
N = B * T
P = N * K
TM = 256
PADTOT = P + E * TM
GT = PADTOT // TM
NSUB = 16
NW = 32
CP = P // NW
GCH = 64
NB = 8
BP = P // NB


def _router_body(x_ref, rw_ref, rb_ref, eid_ref, w_ref):
    logits = jax.lax.dot_general(
        x_ref[...], rw_ref[...], (((1,), (1,)), ((), ())),
        preferred_element_type=jnp.float32,
    ) + rb_ref[...]
    idx = jax.lax.broadcasted_iota(jnp.int32, (N, E), 1)
    m1 = jnp.max(logits, axis=1, keepdims=True)
    a1 = jnp.min(jnp.where(logits == m1, idx, E), axis=1, keepdims=True)
    l2 = jnp.where(idx == a1, -jnp.inf, logits)
    m2 = jnp.max(l2, axis=1, keepdims=True)
    a2 = jnp.min(jnp.where(l2 == m2, idx, E), axis=1, keepdims=True)
    w1 = jax.nn.sigmoid(m1 - m2)
    eid_ref[...] = jnp.concatenate([a1, a2], axis=1)
    w_ref[...] = jnp.concatenate([w1, 1.0 - w1], axis=1)


def _router(xf, router_W, router_b):
    return pl.pallas_call(
        _router_body,
        out_shape=(
            jax.ShapeDtypeStruct((N, K), jnp.int32),
            jax.ShapeDtypeStruct((N, K), jnp.float32),
        ),
        in_specs=[
            pl.BlockSpec((N, DIM), lambda: (0, 0)),
            pl.BlockSpec((E, DIM), lambda: (0, 0)),
            pl.BlockSpec((E,), lambda: (0,)),
        ],
        out_specs=(
            pl.BlockSpec((N, K), lambda: (0, 0)),
            pl.BlockSpec((N, K), lambda: (0, 0)),
        ),
    )(xf, router_W, router_b)


def _bin_body(e_ref, pos_ref, gids_ref, tot_ref, run_ref):
    ph = pl.program_id(0)
    blk = pl.program_id(1)

    eb = e_ref[0]
    lanes8 = jax.lax.broadcasted_iota(jnp.int32, (1, E), 1)
    oh = (eb == lanes8).astype(jnp.float32)

    @pl.when(jnp.logical_and(ph == 0, blk == 0))
    def _init0():
        tot_ref[...] = jnp.zeros((1, 128), jnp.float32)

    @pl.when(ph == 0)
    def _count():
        tot_ref[:, :E] = tot_ref[:, :E] + jnp.sum(oh, axis=0, keepdims=True)
        pos_ref[...] = jnp.zeros((1, BP, 1), jnp.int32)

    @pl.when(jnp.logical_and(ph == 1, blk == 0))
    def _init1():
        run_ref[...] = jnp.zeros((1, 128), jnp.float32)

    @pl.when(ph == 1)
    def _rank():
        tot = tot_ref[:, :E]
        toti = tot.astype(jnp.int32)
        padcnt = ((toti + (TM - 1)) >> 8) << 8
        le8 = jax.lax.broadcasted_iota(jnp.int32, (E, E), 0) <= \
            jax.lax.broadcasted_iota(jnp.int32, (E, E), 1)
        incl = jax.lax.dot_general(
            padcnt.astype(jnp.float32), le8.astype(jnp.float32),
            (((1,), (0,)), ((), ())), preferred_element_type=jnp.float32)
        padoff = incl - padcnt.astype(jnp.float32)

        ir = jax.lax.broadcasted_iota(jnp.int32, (BP, BP), 0)
        ic = jax.lax.broadcasted_iota(jnp.int32, (BP, BP), 1)
        tri = (ic < ir).astype(jnp.float32)
        excl = jax.lax.dot_general(
            tri, oh, (((1,), (0,)), ((), ())),
            preferred_element_type=jnp.float32)

        slot = jnp.sum(oh * (padoff + run_ref[:, :E] + excl),
                       axis=1, keepdims=True)
        pos_ref[...] = slot.astype(jnp.int32).reshape(1, BP, 1)
        run_ref[:, :E] = run_ref[:, :E] + jnp.sum(oh, axis=0, keepdims=True)

        @pl.when(blk == 0)
        def _gids():
            ts = jax.lax.broadcasted_iota(jnp.int32, (1, 64), 1).astype(
                jnp.float32) * TM
            g = jnp.zeros((1, 64), jnp.float32)
            for e in range(E):
                incl_e = jax.lax.dot_general(
                    incl, (lanes8 == e).astype(jnp.float32),
                    (((1,), (1,)), ((), ())),
                    preferred_element_type=jnp.float32)
                g = g + jnp.where(ts >= incl_e, 1.0, 0.0)
                if e == E - 1:
                    ntiles = incl_e * (1.0 / TM)
            g = jnp.minimum(g, E - 1)
            lanes64 = jax.lax.broadcasted_iota(jnp.int32, (1, 64), 1)
            gids_ref[...] = jnp.where(lanes64 == 63, ntiles, g).astype(jnp.int32)


def _binning(eT3):
    return pl.pallas_call(
        _bin_body,
        grid=(2, NB),
        out_shape=(
            jax.ShapeDtypeStruct((NB + 1, BP, 1), jnp.int32),
            jax.ShapeDtypeStruct((1, 64), jnp.int32),
        ),
        in_specs=[pl.BlockSpec((1, BP, 1), lambda ph, blk: (blk, 0, 0))],
        out_specs=(
            pl.BlockSpec((1, BP, 1),
                         lambda ph, blk: (jnp.where(ph == 0, NB, blk), 0, 0)),
            pl.BlockSpec((1, 64), lambda ph, blk: (0, 0)),
        ),
        scratch_shapes=[
            pltpu.VMEM((1, 128), jnp.float32),
            pltpu.VMEM((1, 128), jnp.float32),
        ],
        compiler_params=pltpu.CompilerParams(
            dimension_semantics=("arbitrary", "arbitrary"),
        ),
    )(eT3)


@functools.cache
def _mesh():
    return plsc.VectorSubcoreMesh(core_axis_name="c", subcore_axis_name="s",
                                  num_cores=2, num_subcores=NSUB)


def _sc_dispatch_body(pos_hbm, x_hbm, xdisp_hbm, pos_v, rows0, rows1,
                      lsem0, lsem1, ssem0, ssem1):
    wid = lax.axis_index("s") * 2 + lax.axis_index("c")
    tokbase = (wid % NSUB) * CP
    pltpu.sync_copy(pos_hbm.at[pl.ds(wid * (CP // GCH), CP // GCH)], pos_v)
    rows, lsem, ssem = [rows0, rows1], [lsem0, lsem1], [ssem0, ssem1]
    nch = CP // GCH
    loads, scats = [None] * nch, [None] * nch
    loads[0] = pltpu.async_copy(x_hbm.at[pl.ds(tokbase, GCH)], rows0, lsem0)
    for g in range(nch):
        b = g % 2
        loads[g].wait()
        scats[g] = pltpu.async_copy(rows[b], xdisp_hbm.at[pos_v.at[g]], ssem[b])
        if g + 1 < nch:
            if g >= 1:
                scats[g - 1].wait()
            loads[g + 1] = pltpu.async_copy(
                x_hbm.at[pl.ds(tokbase + (g + 1) * GCH, GCH)],
                rows[1 - b], lsem[1 - b])
    scats[nch - 2].wait()
    scats[nch - 1].wait()


@functools.cache
def _sc_dispatch():
    return pl.kernel(
        _sc_dispatch_body,
        out_type=jax.ShapeDtypeStruct((PADTOT, DIM), jnp.float32),
        mesh=_mesh(),
        compiler_params=pltpu.CompilerParams(needs_layout_passes=False),
        scratch_types=[
            pltpu.VMEM((CP // GCH, GCH), jnp.int32),
            pltpu.VMEM((GCH, DIM), jnp.float32),
            pltpu.VMEM((GCH, DIM), jnp.float32),
            pltpu.SemaphoreType.DMA,
            pltpu.SemaphoreType.DMA,
            pltpu.SemaphoreType.DMA,
            pltpu.SemaphoreType.DMA,
        ],
    )


def _gmm_body(gid_ref, x_ref, w1_ref, b1_ref, w2_ref, b2_ref, y_ref):
    @pl.when(pl.program_id(0) < gid_ref[63])
    def _live():
        xt = x_ref[...]
        y = jnp.zeros((TM, DIM), jnp.float32)
        CHUNK = 2048
        for j0 in range(0, HID, CHUNK):
            a = jnp.dot(xt, w1_ref[0, :, j0:j0 + CHUNK],
                        preferred_element_type=jnp.float32) + b1_ref[0, 0, j0:j0 + CHUNK]
            g = jnp.dot(xt, w1_ref[0, :, HID + j0:HID + j0 + CHUNK],
                        preferred_element_type=jnp.float32) + b1_ref[0, 0, HID + j0:HID + j0 + CHUNK]
            act = (a * jax.nn.sigmoid(a)) * g
            y = y + jnp.dot(act, w2_ref[0, j0:j0 + CHUNK, :],
                            preferred_element_type=jnp.float32)
        y_ref[...] = y + b2_ref[0, 0, :]


def _gmm(gids, xdisp, w1b, b1r, w2b, b2r):
    grid_spec = pltpu.PrefetchScalarGridSpec(
        num_scalar_prefetch=1,
        grid=(GT,),
        in_specs=[
            pl.BlockSpec((TM, DIM), lambda i, g: (i, 0)),
            pl.BlockSpec((1, DIM, INNER), lambda i, g: (g[i], 0, 0)),
            pl.BlockSpec((1, 1, INNER), lambda i, g: (g[i], 0, 0)),
            pl.BlockSpec((1, HID, DIM), lambda i, g: (g[i], 0, 0)),
            pl.BlockSpec((1, 1, DIM), lambda i, g: (g[i], 0, 0)),
        ],
        out_specs=pl.BlockSpec((TM, DIM), lambda i, g: (i, 0)),
    )
    return pl.pallas_call(
        _gmm_body,
        grid_spec=grid_spec,
        out_shape=jax.ShapeDtypeStruct((PADTOT, DIM), jnp.float32),
        compiler_params=pltpu.CompilerParams(
            dimension_semantics=("arbitrary",),
        ),
    )(gids, xdisp, w1b, b1r, w2b, b2r)


def _sc_combine_body(ydisp_hbm, pos_hbm, ysort_hbm, pos_v, rows0, rows1,
                     gsem0, gsem1, psem0, psem1):
    wid = lax.axis_index("s") * 2 + lax.axis_index("c")
    r0 = wid * CP
    pltpu.sync_copy(pos_hbm.at[pl.ds(wid * (CP // GCH), CP // GCH)], pos_v)
    rows, gsem, psem = [rows0, rows1], [gsem0, gsem1], [psem0, psem1]
    nch = CP // GCH
    gets, puts = [None] * nch, [None] * nch
    gets[0] = pltpu.async_copy(ydisp_hbm.at[pos_v.at[0]], rows0, gsem0)
    for g in range(nch):
        b = g % 2
        gets[g].wait()
        puts[g] = pltpu.async_copy(
            rows[b], ysort_hbm.at[pl.ds(r0 + g * GCH, GCH)], psem[b])
        if g + 1 < nch:
            if g >= 1:
                puts[g - 1].wait()
            gets[g + 1] = pltpu.async_copy(
                ydisp_hbm.at[pos_v.at[g + 1]], rows[1 - b], gsem[1 - b])
    puts[nch - 2].wait()
    puts[nch - 1].wait()


@functools.cache
def _sc_combine():
    return pl.kernel(
        _sc_combine_body,
        out_type=jax.ShapeDtypeStruct((P, DIM), jnp.float32),
        mesh=_mesh(),
        compiler_params=pltpu.CompilerParams(needs_layout_passes=False),
        scratch_types=[
            pltpu.VMEM((CP // GCH, GCH), jnp.int32),
            pltpu.VMEM((GCH, DIM), jnp.float32),
            pltpu.VMEM((GCH, DIM), jnp.float32),
            pltpu.SemaphoreType.DMA,
            pltpu.SemaphoreType.DMA,
            pltpu.SemaphoreType.DMA,
            pltpu.SemaphoreType.DMA,
        ],
    )


def _add_body(w_ref, y0_ref, y1_ref, o_ref):
    w = w_ref[...]
    o_ref[...] = w[:, 0:1] * y0_ref[0] + w[:, 1:2] * y1_ref[0]


def _pair_add(wts, ys3):
    NTB = 8
    return pl.pallas_call(
        _add_body,
        grid=(NTB,),
        out_shape=jax.ShapeDtypeStruct((N, DIM), jnp.float32),
        in_specs=[
            pl.BlockSpec((N // NTB, K), lambda i: (i, 0)),
            pl.BlockSpec((1, N // NTB, DIM), lambda i: (0, i, 0)),
            pl.BlockSpec((1, N // NTB, DIM), lambda i: (1, i, 0)),
        ],
        out_specs=pl.BlockSpec((N // NTB, DIM), lambda i: (i, 0)),
    )(wts, ys3, ys3)


@jax.jit
def kernel(x, router_W, router_b, W1, b1, W2, b2):
    xf = x.reshape(N, DIM)
    eids, wts = _router(xf, router_W, router_b)
    eT3 = jnp.transpose(eids).reshape(NB, BP, 1)
    pos3, gids64 = _binning(eT3)
    posf = pos3[:NB].reshape(P)
    xdisp = _sc_dispatch()(posf.reshape(NW * (CP // GCH), GCH), xf)
    ydisp = _gmm(gids64.reshape(64), xdisp,
                 W1, b1.reshape(E, 1, INNER),
                 W2, b2.reshape(E, 1, DIM))
    ysorted = _sc_combine()(ydisp, posf.reshape(NW * (CP // GCH), GCH))
    out = _pair_add(wts, ysorted.reshape(K, N, DIM))
    return out.reshape(B, T, DIM)


# speedup vs baseline: 1.1011x; 1.0397x over previous
import functools

import jax
import jax.numpy as jnp
from jax import lax
from jax.experimental import pallas as pl
from jax.experimental.pallas import tpu as pltpu
from jax.experimental.pallas import tpu_sc as plsc

B, T, DIM = 2, 2048, 768
E, K, HID = 8, 2, 2048
INNER = 2 * HID
N = B * T
P = N * K
TM = 512
TMSH = 9
PADTOT = P + E * TM
GT = PADTOT // TM
NSUB = 16
NW = 32
CP = P // NW
GCH = 64
NB = 8
BP = P // NB


def _router_body(x_ref, rw_ref, rb_ref, eid_ref, w_ref):
    logits = jax.lax.dot_general(
        x_ref[...], rw_ref[...], (((1,), (1,)), ((), ())),
        preferred_element_type=jnp.float32,
    ) + rb_ref[...]
    idx = jax.lax.broadcasted_iota(jnp.int32, (N, E), 1)
    m1 = jnp.max(logits, axis=1, keepdims=True)
    a1 = jnp.min(jnp.where(logits == m1, idx, E), axis=1, keepdims=True)
    l2 = jnp.where(idx == a1, -jnp.inf, logits)
    m2 = jnp.max(l2, axis=1, keepdims=True)
    a2 = jnp.min(jnp.where(l2 == m2, idx, E), axis=1, keepdims=True)
    w1 = jax.nn.sigmoid(m1 - m2)
    eid_ref[...] = jnp.concatenate([a1, a2], axis=1)
    w_ref[...] = jnp.concatenate([w1, 1.0 - w1], axis=1)


def _router(xf, router_W, router_b):
    return pl.pallas_call(
        _router_body,
        out_shape=(
            jax.ShapeDtypeStruct((N, K), jnp.int32),
            jax.ShapeDtypeStruct((N, K), jnp.float32),
        ),
        in_specs=[
            pl.BlockSpec((N, DIM), lambda: (0, 0)),
            pl.BlockSpec((E, DIM), lambda: (0, 0)),
            pl.BlockSpec((E,), lambda: (0,)),
        ],
        out_specs=(
            pl.BlockSpec((N, K), lambda: (0, 0)),
            pl.BlockSpec((N, K), lambda: (0, 0)),
        ),
    )(xf, router_W, router_b)


def _bin_body(e_ref, pos_ref, gids_ref, tot_ref, run_ref):
    ph = pl.program_id(0)
    blk = pl.program_id(1)

    eb = e_ref[0]
    lanes8 = jax.lax.broadcasted_iota(jnp.int32, (1, E), 1)
    oh = (eb == lanes8).astype(jnp.float32)

    @pl.when(jnp.logical_and(ph == 0, blk == 0))
    def _init0():
        tot_ref[...] = jnp.zeros((1, 128), jnp.float32)

    @pl.when(ph == 0)
    def _count():
        tot_ref[:, :E] = tot_ref[:, :E] + jnp.sum(oh, axis=0, keepdims=True)
        pos_ref[...] = jnp.zeros((1, BP, 1), jnp.int32)

    @pl.when(jnp.logical_and(ph == 1, blk == 0))
    def _init1():
        run_ref[...] = jnp.zeros((1, 128), jnp.float32)

    @pl.when(ph == 1)
    def _rank():
        tot = tot_ref[:, :E]
        toti = tot.astype(jnp.int32)
        padcnt = ((toti + (TM - 1)) >> TMSH) << TMSH
        le8 = jax.lax.broadcasted_iota(jnp.int32, (E, E), 0) <= \
            jax.lax.broadcasted_iota(jnp.int32, (E, E), 1)
        incl = jax.lax.dot_general(
            padcnt.astype(jnp.float32), le8.astype(jnp.float32),
            (((1,), (0,)), ((), ())), preferred_element_type=jnp.float32)
        padoff = incl - padcnt.astype(jnp.float32)

        ir = jax.lax.broadcasted_iota(jnp.int32, (BP, BP), 0)
        ic = jax.lax.broadcasted_iota(jnp.int32, (BP, BP), 1)
        tri = (ic < ir).astype(jnp.float32)
        excl = jax.lax.dot_general(
            tri, oh, (((1,), (0,)), ((), ())),
            preferred_element_type=jnp.float32)

        slot = jnp.sum(oh * (padoff + run_ref[:, :E] + excl),
                       axis=1, keepdims=True)
        pos_ref[...] = slot.astype(jnp.int32).reshape(1, BP, 1)
        run_ref[:, :E] = run_ref[:, :E] + jnp.sum(oh, axis=0, keepdims=True)

        @pl.when(blk == 0)
        def _gids():
            ts = jax.lax.broadcasted_iota(jnp.int32, (1, 64), 1).astype(
                jnp.float32) * TM
            g = jnp.zeros((1, 64), jnp.float32)
            for e in range(E):
                incl_e = jax.lax.dot_general(
                    incl, (lanes8 == e).astype(jnp.float32),
                    (((1,), (1,)), ((), ())),
                    preferred_element_type=jnp.float32)
                g = g + jnp.where(ts >= incl_e, 1.0, 0.0)
                if e == E - 1:
                    ntiles = incl_e * (1.0 / TM)
            g = jnp.minimum(g, E - 1)
            lanes64 = jax.lax.broadcasted_iota(jnp.int32, (1, 64), 1)
            gids_ref[...] = jnp.where(lanes64 == 63, ntiles, g).astype(jnp.int32)


def _binning(eT3):
    return pl.pallas_call(
        _bin_body,
        grid=(2, NB),
        out_shape=(
            jax.ShapeDtypeStruct((NB + 1, BP, 1), jnp.int32),
            jax.ShapeDtypeStruct((1, 64), jnp.int32),
        ),
        in_specs=[pl.BlockSpec((1, BP, 1), lambda ph, blk: (blk, 0, 0))],
        out_specs=(
            pl.BlockSpec((1, BP, 1),
                         lambda ph, blk: (jnp.where(ph == 0, NB, blk), 0, 0)),
            pl.BlockSpec((1, 64), lambda ph, blk: (0, 0)),
        ),
        scratch_shapes=[
            pltpu.VMEM((1, 128), jnp.float32),
            pltpu.VMEM((1, 128), jnp.float32),
        ],
        compiler_params=pltpu.CompilerParams(
            dimension_semantics=("arbitrary", "arbitrary"),
        ),
    )(eT3)


@functools.cache
def _mesh():
    return plsc.VectorSubcoreMesh(core_axis_name="c", subcore_axis_name="s",
                                  num_cores=2, num_subcores=NSUB)


def _sc_dispatch_body(pos_hbm, x_hbm, xdisp_hbm, pos_v, rows0, rows1,
                      lsem0, lsem1, ssem0, ssem1):
    wid = lax.axis_index("s") * 2 + lax.axis_index("c")
    tokbase = (wid % NSUB) * CP
    pltpu.sync_copy(pos_hbm.at[pl.ds(wid * (CP // GCH), CP // GCH)], pos_v)
    rows, lsem, ssem = [rows0, rows1], [lsem0, lsem1], [ssem0, ssem1]
    nch = CP // GCH
    loads, scats = [None] * nch, [None] * nch
    loads[0] = pltpu.async_copy(x_hbm.at[pl.ds(tokbase, GCH)], rows0, lsem0)
    for g in range(nch):
        b = g % 2
        loads[g].wait()
        scats[g] = pltpu.async_copy(rows[b], xdisp_hbm.at[pos_v.at[g]], ssem[b])
        if g + 1 < nch:
            if g >= 1:
                scats[g - 1].wait()
            loads[g + 1] = pltpu.async_copy(
                x_hbm.at[pl.ds(tokbase + (g + 1) * GCH, GCH)],
                rows[1 - b], lsem[1 - b])
    scats[nch - 2].wait()
    scats[nch - 1].wait()


@functools.cache
def _sc_dispatch():
    return pl.kernel(
        _sc_dispatch_body,
        out_type=jax.ShapeDtypeStruct((PADTOT, DIM), jnp.float32),
        mesh=_mesh(),
        compiler_params=pltpu.CompilerParams(needs_layout_passes=False),
        scratch_types=[
            pltpu.VMEM((CP // GCH, GCH), jnp.int32),
            pltpu.VMEM((GCH, DIM), jnp.float32),
            pltpu.VMEM((GCH, DIM), jnp.float32),
            pltpu.SemaphoreType.DMA,
            pltpu.SemaphoreType.DMA,
            pltpu.SemaphoreType.DMA,
            pltpu.SemaphoreType.DMA,
        ],
    )


def _gmm_body(gid_ref, x_ref, w1_ref, b1_ref, w2_ref, b2_ref, y_ref):
    @pl.when(pl.program_id(0) < gid_ref[63])
    def _live():
        xt = x_ref[...]
        y = jnp.zeros((TM, DIM), jnp.float32)
        CHUNK = 2048
        for j0 in range(0, HID, CHUNK):
            a = jnp.dot(xt, w1_ref[0, :, j0:j0 + CHUNK],
                        preferred_element_type=jnp.float32) + b1_ref[0, 0, j0:j0 + CHUNK]
            g = jnp.dot(xt, w1_ref[0, :, HID + j0:HID + j0 + CHUNK],
                        preferred_element_type=jnp.float32) + b1_ref[0, 0, HID + j0:HID + j0 + CHUNK]
            act = (a * jax.nn.sigmoid(a)) * g
            y = y + jnp.dot(act, w2_ref[0, j0:j0 + CHUNK, :],
                            preferred_element_type=jnp.float32)
        y_ref[...] = y + b2_ref[0, 0, :]


def _gmm(gids, xdisp, w1b, b1r, w2b, b2r):
    grid_spec = pltpu.PrefetchScalarGridSpec(
        num_scalar_prefetch=1,
        grid=(GT,),
        in_specs=[
            pl.BlockSpec((TM, DIM), lambda i, g: (i, 0)),
            pl.BlockSpec((1, DIM, INNER), lambda i, g: (g[i], 0, 0)),
            pl.BlockSpec((1, 1, INNER), lambda i, g: (g[i], 0, 0)),
            pl.BlockSpec((1, HID, DIM), lambda i, g: (g[i], 0, 0)),
            pl.BlockSpec((1, 1, DIM), lambda i, g: (g[i], 0, 0)),
        ],
        out_specs=pl.BlockSpec((TM, DIM), lambda i, g: (i, 0)),
    )
    return pl.pallas_call(
        _gmm_body,
        grid_spec=grid_spec,
        out_shape=jax.ShapeDtypeStruct((PADTOT, DIM), jnp.float32),
        compiler_params=pltpu.CompilerParams(
            dimension_semantics=("arbitrary",),
        ),
    )(gids, xdisp, w1b, b1r, w2b, b2r)


def _sc_combine_body(ydisp_hbm, pos_hbm, ysort_hbm, pos_v, rows0, rows1,
                     gsem0, gsem1, psem0, psem1):
    wid = lax.axis_index("s") * 2 + lax.axis_index("c")
    r0 = wid * CP
    pltpu.sync_copy(pos_hbm.at[pl.ds(wid * (CP // GCH), CP // GCH)], pos_v)
    rows, gsem, psem = [rows0, rows1], [gsem0, gsem1], [psem0, psem1]
    nch = CP // GCH
    gets, puts = [None] * nch, [None] * nch
    gets[0] = pltpu.async_copy(ydisp_hbm.at[pos_v.at[0]], rows0, gsem0)
    for g in range(nch):
        b = g % 2
        gets[g].wait()
        puts[g] = pltpu.async_copy(
            rows[b], ysort_hbm.at[pl.ds(r0 + g * GCH, GCH)], psem[b])
        if g + 1 < nch:
            if g >= 1:
                puts[g - 1].wait()
            gets[g + 1] = pltpu.async_copy(
                ydisp_hbm.at[pos_v.at[g + 1]], rows[1 - b], gsem[1 - b])
    puts[nch - 2].wait()
    puts[nch - 1].wait()


@functools.cache
def _sc_combine():
    return pl.kernel(
        _sc_combine_body,
        out_type=jax.ShapeDtypeStruct((P, DIM), jnp.float32),
        mesh=_mesh(),
        compiler_params=pltpu.CompilerParams(needs_layout_passes=False),
        scratch_types=[
            pltpu.VMEM((CP // GCH, GCH), jnp.int32),
            pltpu.VMEM((GCH, DIM), jnp.float32),
            pltpu.VMEM((GCH, DIM), jnp.float32),
            pltpu.SemaphoreType.DMA,
            pltpu.SemaphoreType.DMA,
            pltpu.SemaphoreType.DMA,
            pltpu.SemaphoreType.DMA,
        ],
    )


def _add_body(w_ref, y0_ref, y1_ref, o_ref):
    w = w_ref[...]
    o_ref[...] = w[:, 0:1] * y0_ref[0] + w[:, 1:2] * y1_ref[0]


def _pair_add(wts, ys3):
    NTB = 8
    return pl.pallas_call(
        _add_body,
        grid=(NTB,),
        out_shape=jax.ShapeDtypeStruct((N, DIM), jnp.float32),
        in_specs=[
            pl.BlockSpec((N // NTB, K), lambda i: (i, 0)),
            pl.BlockSpec((1, N // NTB, DIM), lambda i: (0, i, 0)),
            pl.BlockSpec((1, N // NTB, DIM), lambda i: (1, i, 0)),
        ],
        out_specs=pl.BlockSpec((N // NTB, DIM), lambda i: (i, 0)),
    )(wts, ys3, ys3)


@jax.jit
def kernel(x, router_W, router_b, W1, b1, W2, b2):
    xf = x.reshape(N, DIM)
    eids, wts = _router(xf, router_W, router_b)
    eT3 = jnp.transpose(eids).reshape(NB, BP, 1)
    pos3, gids64 = _binning(eT3)
    posf = pos3[:NB].reshape(P)
    xdisp = _sc_dispatch()(posf.reshape(NW * (CP // GCH), GCH), xf)
    ydisp = _gmm(gids64.reshape(64), xdisp,
                 W1, b1.reshape(E, 1, INNER),
                 W2, b2.reshape(E, 1, DIM))
    ysorted = _sc_combine()(ydisp, posf.reshape(NW * (CP // GCH), GCH))
    out = _pair_add(wts, ysorted.reshape(K, N, DIM))
    return out.reshape(B, T, DIM)
